# baseline (device time: 60518 ns/iter reference)
import os

import jax
import jax.numpy as jnp
from jax import lax
from jax.experimental import pallas as pl
from jax.experimental.pallas import tpu as pltpu

B, QL, H, D = 4, 32, 8, 128
BH = B * H
NZ = 4
G = 2
BG = B // G
BHG = BG * H
C = 512
SCALE = D ** -0.5

_SKIP_COMM = os.environ.get("KERNEL_SKIP_COMM") == "1"


def kernel(Q, K, V):
    SK = K.shape[1]
    NCG = SK // C

    def body(q_ref, k_ref, v_ref, out_ref,
             o_scr, m_scr, l_scr, comm_o, comm_s,
             send_o, recv_o, send_s, recv_s):
        g = pl.program_id(0)
        n = pl.program_id(1)
        rows = pl.ds(g * BHG, BHG)

        @pl.when((g == 0) & (n == 0))
        def _init():
            m_scr[...] = jnp.full((BH, QL, 1), -1e30, jnp.float32)
            l_scr[...] = jnp.zeros((BH, QL, 1), jnp.float32)
            o_scr[...] = jnp.zeros((BH, QL, D), jnp.float32)

        q = jnp.transpose(q_ref[...].astype(jnp.bfloat16), (0, 2, 1, 3))
        q = q.reshape(BHG, QL, D)
        k = jnp.transpose(k_ref[...].astype(jnp.bfloat16), (0, 2, 1, 3))
        k = k.reshape(BHG, C, D)
        v = jnp.transpose(v_ref[...].astype(jnp.bfloat16), (0, 2, 1, 3))
        v = v.reshape(BHG, C, D)

        s = lax.dot_general(q, k, (((2,), (2,)), ((0,), (0,))),
                            preferred_element_type=jnp.float32) * SCALE
        m_prev = m_scr[rows]
        m_new = jnp.maximum(m_prev, jnp.max(s, axis=2, keepdims=True))
        alpha = jnp.exp(m_prev - m_new)
        p = jnp.exp(s - m_new)
        l_new = l_scr[rows] * alpha + jnp.sum(p, axis=2, keepdims=True)
        o_new = o_scr[rows] * alpha + lax.dot_general(
            p.astype(jnp.bfloat16), v, (((2,), (1,)), ((0,), (0,))),
            preferred_element_type=jnp.float32)
        m_scr[rows] = m_new
        l_scr[rows] = l_new
        o_scr[rows] = o_new

        if _SKIP_COMM:
            @pl.when((g == G - 1) & (n == NCG - 1))
            def _skip():
                out_ref[...] = jnp.transpose(
                    (o_scr[...] / l_scr[...]).reshape(B, H, QL, D),
                    (0, 2, 1, 3))
            return

        my_x = lax.axis_index("x")
        my_y = lax.axis_index("y")
        my_z = lax.axis_index("z")

        def make_rdmas(grp):
            rd = []
            for d_ in range(1, NZ):
                dst = (my_x, my_y, (my_z + d_) % NZ)
                rd.append(pltpu.make_async_remote_copy(
                    src_ref=comm_o.at[grp, 0],
                    dst_ref=comm_o.at[grp, d_],
                    send_sem=send_o.at[grp, d_ - 1],
                    recv_sem=recv_o.at[grp, d_ - 1],
                    device_id=dst,
                    device_id_type=pl.DeviceIdType.MESH,
                ))
                rd.append(pltpu.make_async_remote_copy(
                    src_ref=comm_s.at[grp, 0],
                    dst_ref=comm_s.at[grp, d_],
                    send_sem=send_s.at[grp, d_ - 1],
                    recv_sem=recv_s.at[grp, d_ - 1],
                    device_id=dst,
                    device_id_type=pl.DeviceIdType.MESH,
                ))
            return rd

        def merge_group(grp, grp_rows):
            M = m_scr[grp_rows]
            L = l_scr[grp_rows]
            O = o_scr[grp_rows]
            for j in range(1, NZ):
                mj = comm_s[grp, j, 0][..., None]
                lj = comm_s[grp, j, 1][..., None]
                oj = comm_o[grp, j].astype(jnp.float32)
                Mn = jnp.maximum(M, mj)
                a = jnp.exp(M - Mn)
                bfac = jnp.exp(mj - Mn)
                O = O * a + oj * bfac
                L = L * a + lj * bfac
                M = Mn
            o_scr[grp_rows] = O / L

        @pl.when((g == 0) & (n == NCG - 1))
        def _send_g0():
            comm_o[0, 0] = o_scr[pl.ds(0, BHG)].astype(jnp.bfloat16)
            comm_s[0, 0, 0] = m_scr[pl.ds(0, BHG)].reshape(BHG, QL)
            comm_s[0, 0, 1] = l_scr[pl.ds(0, BHG)].reshape(BHG, QL)

            barrier = pltpu.get_barrier_semaphore()
            for d_ in range(1, NZ):
                nbr = (my_x, my_y, (my_z + d_) % NZ)
                pl.semaphore_signal(barrier, inc=1, device_id=nbr,
                                    device_id_type=pl.DeviceIdType.MESH)
            pl.semaphore_wait(barrier, NZ - 1)

            for r in make_rdmas(0):
                r.start()

        @pl.when((g == G - 1) & (n == NCG - 1))
        def _finish():
            comm_o[1, 0] = o_scr[pl.ds(BHG, BHG)].astype(jnp.bfloat16)
            comm_s[1, 0, 0] = m_scr[pl.ds(BHG, BHG)].reshape(BHG, QL)
            comm_s[1, 0, 1] = l_scr[pl.ds(BHG, BHG)].reshape(BHG, QL)
            for r in make_rdmas(1):
                r.start()

            for r in make_rdmas(0):
                r.wait()
            merge_group(0, pl.ds(0, BHG))

            for r in make_rdmas(1):
                r.wait()
            merge_group(1, pl.ds(BHG, BHG))

            out_ref[...] = jnp.transpose(
                o_scr[...].reshape(B, H, QL, D), (0, 2, 1, 3))

    return pl.pallas_call(
        body,
        grid=(G, NCG),
        in_specs=[
            pl.BlockSpec((BG, QL, H, D), lambda g, n: (g, 0, 0, 0)),
            pl.BlockSpec((BG, C, H, D), lambda g, n: (g, n, 0, 0)),
            pl.BlockSpec((BG, C, H, D), lambda g, n: (g, n, 0, 0)),
        ],
        out_specs=pl.BlockSpec((B, QL, H, D), lambda g, n: (0, 0, 0, 0)),
        out_shape=jax.ShapeDtypeStruct((B, QL, H, D), jnp.float32),
        scratch_shapes=[
            pltpu.VMEM((BH, QL, D), jnp.float32),
            pltpu.VMEM((BH, QL, 1), jnp.float32),
            pltpu.VMEM((BH, QL, 1), jnp.float32),
            pltpu.VMEM((G, NZ, BHG, QL, D), jnp.bfloat16),
            pltpu.VMEM((G, NZ, 2, BHG, QL), jnp.float32),
            pltpu.SemaphoreType.DMA((G, NZ - 1)),
            pltpu.SemaphoreType.DMA((G, NZ - 1)),
            pltpu.SemaphoreType.DMA((G, NZ - 1)),
            pltpu.SemaphoreType.DMA((G, NZ - 1)),
        ],
        **({} if _SKIP_COMM else
           dict(compiler_params=pltpu.CompilerParams(collective_id=0))),
    )(Q, K, V)


# device time: 42382 ns/iter; 1.4279x vs baseline; 1.4279x over previous
import os

import jax
import jax.numpy as jnp
from jax import lax
from jax.experimental import pallas as pl
from jax.experimental.pallas import tpu as pltpu

B, QL, H, D = 4, 32, 8, 128
BH = B * H
NZ = 4
G = 2
BG = B // G
BHG = BG * H
C = 512
SCALE = D ** -0.5

_SKIP_COMM = os.environ.get("KERNEL_SKIP_COMM") == "1"
_STREAM_ONLY = os.environ.get("KERNEL_STREAM_ONLY") == "1"


def kernel(Q, K, V):
    SK = K.shape[1]
    NCG = SK // C

    def body(q_ref, k_ref, v_ref, out_ref,
             o_scr, m_scr, l_scr, comm_o, comm_s,
             send_o, recv_o, send_s, recv_s):
        g = pl.program_id(0)
        n = pl.program_id(1)
        rows = pl.ds(g * BHG, BHG)

        @pl.when((g == 0) & (n == 0))
        def _init():
            m_scr[...] = jnp.full((BH, QL, 1), -1e30, jnp.float32)
            l_scr[...] = jnp.zeros((BH, QL, 1), jnp.float32)
            o_scr[...] = jnp.zeros((BH, QL, D), jnp.float32)

        if _STREAM_ONLY:
            @pl.when((g == G - 1) & (n == NCG - 1))
            def _stream_out():
                out_ref[...] = (jnp.zeros((B, QL, H, D), jnp.float32)
                                + k_ref[0, 0, 0, 0] + v_ref[0, 0, 0, 0])
            return

        q = jnp.transpose(q_ref[...].astype(jnp.bfloat16), (0, 2, 1, 3))
        q = q.reshape(BHG, QL, D)
        k = jnp.transpose(k_ref[...].astype(jnp.bfloat16), (0, 2, 1, 3))
        k = k.reshape(BHG, C, D)
        v = jnp.transpose(v_ref[...].astype(jnp.bfloat16), (0, 2, 1, 3))
        v = v.reshape(BHG, C, D)

        s = lax.dot_general(q, k, (((2,), (2,)), ((0,), (0,))),
                            preferred_element_type=jnp.float32) * SCALE
        m_prev = m_scr[rows]
        m_new = jnp.maximum(m_prev, jnp.max(s, axis=2, keepdims=True))
        alpha = jnp.exp(m_prev - m_new)
        p = jnp.exp(s - m_new)
        l_new = l_scr[rows] * alpha + jnp.sum(p, axis=2, keepdims=True)
        o_new = o_scr[rows] * alpha + lax.dot_general(
            p.astype(jnp.bfloat16), v, (((2,), (1,)), ((0,), (0,))),
            preferred_element_type=jnp.float32)
        m_scr[rows] = m_new
        l_scr[rows] = l_new
        o_scr[rows] = o_new

        if _SKIP_COMM:
            @pl.when((g == G - 1) & (n == NCG - 1))
            def _skip():
                out_ref[...] = jnp.transpose(
                    (o_scr[...] / l_scr[...]).reshape(B, H, QL, D),
                    (0, 2, 1, 3))
            return

        my_x = lax.axis_index("x")
        my_y = lax.axis_index("y")
        my_z = lax.axis_index("z")

        def make_rdmas(grp):
            rd = []
            for d_ in range(1, NZ):
                dst = (my_x, my_y, (my_z + d_) % NZ)
                rd.append(pltpu.make_async_remote_copy(
                    src_ref=comm_o.at[grp, 0],
                    dst_ref=comm_o.at[grp, d_],
                    send_sem=send_o.at[grp, d_ - 1],
                    recv_sem=recv_o.at[grp, d_ - 1],
                    device_id=dst,
                    device_id_type=pl.DeviceIdType.MESH,
                ))
                rd.append(pltpu.make_async_remote_copy(
                    src_ref=comm_s.at[grp, 0],
                    dst_ref=comm_s.at[grp, d_],
                    send_sem=send_s.at[grp, d_ - 1],
                    recv_sem=recv_s.at[grp, d_ - 1],
                    device_id=dst,
                    device_id_type=pl.DeviceIdType.MESH,
                ))
            return rd

        def merge_group(grp, grp_rows):
            M = m_scr[grp_rows]
            L = l_scr[grp_rows]
            O = o_scr[grp_rows]
            for j in range(1, NZ):
                mj = comm_s[grp, j, 0][..., None]
                lj = comm_s[grp, j, 1][..., None]
                oj = comm_o[grp, j].astype(jnp.float32)
                Mn = jnp.maximum(M, mj)
                a = jnp.exp(M - Mn)
                bfac = jnp.exp(mj - Mn)
                O = O * a + oj * bfac
                L = L * a + lj * bfac
                M = Mn
            o_scr[grp_rows] = O / L

        @pl.when((g == 0) & (n == NCG - 1))
        def _send_g0():
            comm_o[0, 0] = o_scr[pl.ds(0, BHG)].astype(jnp.bfloat16)
            comm_s[0, 0, 0] = m_scr[pl.ds(0, BHG)].reshape(BHG, QL)
            comm_s[0, 0, 1] = l_scr[pl.ds(0, BHG)].reshape(BHG, QL)

            barrier = pltpu.get_barrier_semaphore()
            for d_ in range(1, NZ):
                nbr = (my_x, my_y, (my_z + d_) % NZ)
                pl.semaphore_signal(barrier, inc=1, device_id=nbr,
                                    device_id_type=pl.DeviceIdType.MESH)
            pl.semaphore_wait(barrier, NZ - 1)

            for r in make_rdmas(0):
                r.start()

        @pl.when((g == G - 1) & (n == NCG - 1))
        def _finish():
            comm_o[1, 0] = o_scr[pl.ds(BHG, BHG)].astype(jnp.bfloat16)
            comm_s[1, 0, 0] = m_scr[pl.ds(BHG, BHG)].reshape(BHG, QL)
            comm_s[1, 0, 1] = l_scr[pl.ds(BHG, BHG)].reshape(BHG, QL)
            for r in make_rdmas(1):
                r.start()

            for r in make_rdmas(0):
                r.wait()
            merge_group(0, pl.ds(0, BHG))

            for r in make_rdmas(1):
                r.wait()
            merge_group(1, pl.ds(BHG, BHG))

            out_ref[...] = jnp.transpose(
                o_scr[...].reshape(B, H, QL, D), (0, 2, 1, 3))

    return pl.pallas_call(
        body,
        grid=(G, NCG),
        in_specs=[
            pl.BlockSpec((BG, QL, H, D), lambda g, n: (g, 0, 0, 0)),
            pl.BlockSpec((BG, C, H, D), lambda g, n: (g, n, 0, 0)),
            pl.BlockSpec((BG, C, H, D), lambda g, n: (g, n, 0, 0)),
        ],
        out_specs=pl.BlockSpec((B, QL, H, D), lambda g, n: (0, 0, 0, 0)),
        out_shape=jax.ShapeDtypeStruct((B, QL, H, D), jnp.float32),
        scratch_shapes=[
            pltpu.VMEM((BH, QL, D), jnp.float32),
            pltpu.VMEM((BH, QL, 1), jnp.float32),
            pltpu.VMEM((BH, QL, 1), jnp.float32),
            pltpu.VMEM((G, NZ, BHG, QL, D), jnp.bfloat16),
            pltpu.VMEM((G, NZ, 2, BHG, QL), jnp.float32),
            pltpu.SemaphoreType.DMA((G, NZ - 1)),
            pltpu.SemaphoreType.DMA((G, NZ - 1)),
            pltpu.SemaphoreType.DMA((G, NZ - 1)),
            pltpu.SemaphoreType.DMA((G, NZ - 1)),
        ],
        **({} if (_SKIP_COMM or _STREAM_ONLY) else
           dict(compiler_params=pltpu.CompilerParams(collective_id=0))),
    )(Q, K, V)
